# baseline (speedup 1.0000x reference)
import functools

import jax
import jax.numpy as jnp
from jax import lax
from jax.experimental import pallas as pl
from jax.experimental.pallas import tpu as pltpu
import jax.experimental.pallas.tpu_sc as plsc

N_ROWS = 50000
LATENT = 128
CHUNK_W = 32
N_CHUNKS = LATENT // CHUNK_W
NNZ = 400000
N_SUB = 16
BLK = 256
SUB = 128
N_BLK = 102
PER_TEC = N_BLK * BLK
NNZP = PER_TEC * N_SUB
ACC_ROWS = 50176
ROWS_PER_TEC = ACC_ROWS // N_SUB
ZROWS = 112
G_RING = 3
IB_RING = 6
N_TBLK = N_SUB * N_BLK


def _sc_spmm(u2, i2, colsb, rowsb, valsb):
    mesh = plsc.VectorSubcoreMesh(core_axis_name="c", subcore_axis_name="s")

    @functools.partial(
        pl.kernel,
        mesh=mesh,
        compiler_params=pltpu.CompilerParams(use_tc_tiling_on_sc=False,
                                             needs_layout_passes=False),
        out_type=jax.ShapeDtypeStruct((2, ACC_ROWS, LATENT), jnp.bfloat16),
        scratch_types=[
            pltpu.VMEM_SHARED((ACC_ROWS, CHUNK_W), jnp.bfloat16),
            pltpu.VMEM((G_RING, BLK, CHUNK_W), jnp.float32),
            pltpu.VMEM((G_RING, BLK, CHUNK_W), jnp.bfloat16),
            pltpu.VMEM((IB_RING, 4, SUB), jnp.int32),
            pltpu.VMEM((IB_RING, 2, SUB), jnp.float32),
        ] + [pltpu.SemaphoreType.DMA] * (G_RING + G_RING + IB_RING + 1),
    )
    def k(u2_hbm, i2_hbm, colsb_hbm, rowsb_hbm, valsb_hbm, out_hbm,
          acc, g, gb, ib, vb,
          gs0, gs1, gs2, ss0, ss1, ss2,
          is0, is1, is2, is3, is4, is5, zsem):
        gsem = [gs0, gs1, gs2]
        ssem = [ss0, ss1, ss2]
        ibsem = [is0, is1, is2, is3, is4, is5]
        c = lax.axis_index("c")
        s = lax.axis_index("s")

        zero32 = jnp.zeros((CHUNK_W,), jnp.bfloat16)
        tblk0 = s * N_BLK

        def issue_ib(bi, slot):
            pltpu.async_copy(colsb_hbm.at[c, tblk0 + bi],
                             ib.at[slot, pl.ds(0, 2)], ibsem[slot])
            pltpu.async_copy(rowsb_hbm.at[c, tblk0 + bi],
                             ib.at[slot, pl.ds(2, 2)], ibsem[slot])
            pltpu.async_copy(valsb_hbm.at[c, tblk0 + bi],
                             vb.at[slot], ibsem[slot])

        def wait_ib(slot):
            pltpu.make_async_copy(colsb_hbm.at[c, tblk0],
                                  ib.at[slot, pl.ds(0, 2)], ibsem[slot]).wait()
            pltpu.make_async_copy(rowsb_hbm.at[c, tblk0],
                                  ib.at[slot, pl.ds(2, 2)], ibsem[slot]).wait()
            pltpu.make_async_copy(valsb_hbm.at[c, tblk0],
                                  vb.at[slot], ibsem[slot]).wait()

        def prep_and_gather(bi, kslot, gslot, p):
            del bi
            wait_ib(kslot)
            if p != 0:
                off = jnp.full((16,), p, jnp.int32)
                for jj in range(2):
                    for t2 in range(SUB // 16):
                        ib[kslot, jj, pl.ds(16 * t2, 16)] = (
                            ib[kslot, jj, pl.ds(16 * t2, 16)] + off)
            @pl.when(c == 0)
            def _():
                for jj in range(2):
                    pltpu.async_copy(u2_hbm.at[ib.at[kslot, jj]],
                                     g.at[gslot, pl.ds(SUB * jj, SUB)],
                                     gsem[gslot])
            @pl.when(c == 1)
            def _():
                for jj in range(2):
                    pltpu.async_copy(i2_hbm.at[ib.at[kslot, jj]],
                                     g.at[gslot, pl.ds(SUB * jj, SUB)],
                                     gsem[gslot])

        def wait_gather(gslot):
            for jj in range(2):
                pltpu.make_async_copy(u2_hbm.at[ib.at[0, 0]],
                                      g.at[gslot, pl.ds(SUB * jj, SUB)],
                                      gsem[gslot]).wait()

        def mul_block(kslot, gslot):
            def mul_body(t, _):
                for jj in range(2):
                    vv = vb[kslot, jj, pl.ds(t * 16, 16)]
                    for h in range(2):
                        rbase = jj * SUB + t * 16 + h * 8
                        loads = []
                        for l in range(8):
                            loads.append((g[gslot, rbase + l, pl.ds(0, 16)],
                                          g[gslot, rbase + l, pl.ds(16, 16)]))
                        for l in range(8):
                            v = vv[h * 8 + l]
                            a, b2 = loads[l]
                            gb[gslot, rbase + l, :] = plsc.pack(
                                a * v, b2 * v,
                                format=plsc.PackFormat.INTERLEAVED)
                return 0
            lax.fori_loop(0, SUB // 16, mul_body, 0)

        def issue_scatter(kslot, gslot):
            for jj in range(2):
                pltpu.async_copy(gb.at[gslot, pl.ds(SUB * jj, SUB)],
                                 acc.at[ib.at[kslot, 2 + jj]],
                                 ssem[gslot], add=True)

        def wait_scatter(gslot):
            for jj in range(2):
                pltpu.make_async_copy(gb.at[gslot, pl.ds(SUB * jj, SUB)],
                                      acc.at[ib.at[0, 2]],
                                      ssem[gslot]).wait()

        for p in range(N_CHUNKS):
            def zb_body(r, _):
                gb[0, r, :] = zero32
                return 0
            lax.fori_loop(0, ZROWS, zb_body, 0)
            zcps = [
                pltpu.async_copy(
                    gb.at[0, pl.ds(0, ZROWS)],
                    acc.at[pl.ds(s * ROWS_PER_TEC + t * ZROWS, ZROWS)], zsem)
                for t in range(ROWS_PER_TEC // ZROWS)
            ]
            for cp in zcps:
                cp.wait()

            for b0 in range(4):
                issue_ib(jnp.int32(b0), b0)
            prep_and_gather(jnp.int32(0), 0, 0, p)
            prep_and_gather(jnp.int32(1), 1, 1, p)
            plsc.subcore_barrier()

            def iter_body(i, _):
                for kk in range(6):
                    j = 6 * i + kk
                    gslot = kk % 3
                    if kk in (0, 1):
                        issue_ib(j + 4, (kk + 4) % 6)
                    else:
                        @pl.when(i < (N_BLK // 6 - 1))
                        def _():
                            issue_ib(j + 4, (kk + 4) % 6)
                    if kk == 0:
                        @pl.when(i > 0)
                        def _():
                            wait_scatter((kk + 2) % 3)
                    else:
                        wait_scatter((kk + 2) % 3)
                    if kk < 4:
                        prep_and_gather(j + 2, (kk + 2) % 6, (kk + 2) % 3, p)
                    else:
                        @pl.when(i < (N_BLK // 6 - 1))
                        def _():
                            prep_and_gather(j + 2, (kk + 2) % 6, (kk + 2) % 3, p)
                    wait_gather(gslot)
                    mul_block(kk, gslot)
                    issue_scatter(kk, gslot)
                return 0

            lax.fori_loop(0, N_BLK // 6, iter_body, 0)
            wait_scatter(2)
            plsc.subcore_barrier()

            pltpu.sync_copy(
                acc.at[pl.ds(s * ROWS_PER_TEC, ROWS_PER_TEC)],
                out_hbm.at[c, pl.ds(s * ROWS_PER_TEC, ROWS_PER_TEC),
                           pl.ds(CHUNK_W * p, CHUNK_W)])
            plsc.subcore_barrier()

    return k(u2, i2, colsb, rowsb, valsb)


def kernel(users_emb, items_emb, u_rows, u_cols, u_vals,
           i_rows, i_cols, i_vals, g_rows, g_cols, g_vals):
    del g_rows, g_cols, g_vals
    pad = NNZP - NNZ
    u2 = users_emb.reshape(N_CHUNKS * N_ROWS, CHUNK_W)
    i2 = items_emb.reshape(N_CHUNKS * N_ROWS, CHUNK_W)
    colsb = jnp.stack([jnp.pad(u_cols, (0, pad)),
                       jnp.pad(i_cols, (0, pad))]
                      ).reshape(2, N_TBLK, 2, SUB) * N_CHUNKS
    rowsb = jnp.stack([jnp.pad(u_rows, (0, pad)),
                       jnp.pad(i_rows, (0, pad))]).reshape(2, N_TBLK, 2, SUB)
    valsb = jnp.stack([jnp.pad(u_vals, (0, pad)),
                       jnp.pad(i_vals, (0, pad))]).reshape(2, N_TBLK, 2, SUB)

    bigo = _sc_spmm(u2, i2, colsb, rowsb, valsb)
    bigo = (bigo.reshape(2, ACC_ROWS, N_CHUNKS, 16, 2)
            .transpose(0, 1, 2, 4, 3)
            .reshape(2, ACC_ROWS, LATENT).astype(jnp.float32))
    return (bigo[0, :N_ROWS], bigo[1, :N_ROWS])


# speedup vs baseline: 1.1912x; 1.1912x over previous
import functools

import jax
import jax.numpy as jnp
from jax import lax
from jax.experimental import pallas as pl
from jax.experimental.pallas import tpu as pltpu
import jax.experimental.pallas.tpu_sc as plsc

N_ROWS = 50000
LATENT = 128
CHUNK_W = 32
N_CHUNKS = LATENT // CHUNK_W
NNZ = 400000
N_SUB = 16
BLK = 256
SUB = 128
N_BLK = 102
PER_TEC = N_BLK * BLK
NNZP = PER_TEC * N_SUB
ACC_ROWS = 50176
ROWS_PER_TEC = ACC_ROWS // N_SUB
ZROWS = 112
G_RING = 3
IB_RING = 6
N_TBLK = N_SUB * N_BLK


def _sc_spmm(u2, i2, colsb, rowsb, valsb):
    mesh = plsc.VectorSubcoreMesh(core_axis_name="c", subcore_axis_name="s")

    @functools.partial(
        pl.kernel,
        mesh=mesh,
        compiler_params=pltpu.CompilerParams(use_tc_tiling_on_sc=False,
                                             needs_layout_passes=False),
        out_type=jax.ShapeDtypeStruct((2, ACC_ROWS, LATENT), jnp.float32),
        scratch_types=[
            pltpu.VMEM_SHARED((ACC_ROWS, CHUNK_W), jnp.float32),
            pltpu.VMEM((G_RING, BLK, CHUNK_W), jnp.float32),
            pltpu.VMEM((IB_RING, 4, SUB), jnp.int32),
            pltpu.VMEM((IB_RING, 2, SUB), jnp.float32),
        ] + [pltpu.SemaphoreType.DMA] * (G_RING + G_RING + IB_RING + 1),
    )
    def k(u2_hbm, i2_hbm, colsb_hbm, rowsb_hbm, valsb_hbm, out_hbm,
          acc, g, ib, vb,
          gs0, gs1, gs2, ss0, ss1, ss2,
          is0, is1, is2, is3, is4, is5, zsem):
        gsem = [gs0, gs1, gs2]
        ssem = [ss0, ss1, ss2]
        ibsem = [is0, is1, is2, is3, is4, is5]
        c = lax.axis_index("c")
        s = lax.axis_index("s")

        zero16 = jnp.zeros((16,), jnp.float32)
        tblk0 = s * N_BLK

        def issue_ib(bi, slot):
            pltpu.async_copy(colsb_hbm.at[c, tblk0 + bi],
                             ib.at[slot, pl.ds(0, 2)], ibsem[slot])
            pltpu.async_copy(rowsb_hbm.at[c, tblk0 + bi],
                             ib.at[slot, pl.ds(2, 2)], ibsem[slot])
            pltpu.async_copy(valsb_hbm.at[c, tblk0 + bi],
                             vb.at[slot], ibsem[slot])

        def wait_ib(slot):
            pltpu.make_async_copy(colsb_hbm.at[c, tblk0],
                                  ib.at[slot, pl.ds(0, 2)], ibsem[slot]).wait()
            pltpu.make_async_copy(rowsb_hbm.at[c, tblk0],
                                  ib.at[slot, pl.ds(2, 2)], ibsem[slot]).wait()
            pltpu.make_async_copy(valsb_hbm.at[c, tblk0],
                                  vb.at[slot], ibsem[slot]).wait()

        def prep_and_gather(bi, kslot, gslot, p):
            del bi
            wait_ib(kslot)
            if p != 0:
                off = jnp.full((16,), p, jnp.int32)
                for jj in range(2):
                    for t2 in range(SUB // 16):
                        ib[kslot, jj, pl.ds(16 * t2, 16)] = (
                            ib[kslot, jj, pl.ds(16 * t2, 16)] + off)
            @pl.when(c == 0)
            def _():
                for jj in range(2):
                    pltpu.async_copy(u2_hbm.at[ib.at[kslot, jj]],
                                     g.at[gslot, pl.ds(SUB * jj, SUB)],
                                     gsem[gslot])
            @pl.when(c == 1)
            def _():
                for jj in range(2):
                    pltpu.async_copy(i2_hbm.at[ib.at[kslot, jj]],
                                     g.at[gslot, pl.ds(SUB * jj, SUB)],
                                     gsem[gslot])

        def wait_gather_sub(gslot, jj):
            pltpu.make_async_copy(u2_hbm.at[ib.at[0, 0]],
                                  g.at[gslot, pl.ds(SUB * jj, SUB)],
                                  gsem[gslot]).wait()

        def mul_sub(kslot, gslot, jj):
            def mul_body(t, _):
                vv = vb[kslot, jj, pl.ds(t * 16, 16)]
                for h in range(2):
                    rbase = jj * SUB + t * 16 + h * 8
                    loads = []
                    for l in range(8):
                        loads.append((g[gslot, rbase + l, pl.ds(0, 16)],
                                      g[gslot, rbase + l, pl.ds(16, 16)]))
                    for l in range(8):
                        v = vv[h * 8 + l]
                        a, b2 = loads[l]
                        g[gslot, rbase + l, pl.ds(0, 16)] = a * v
                        g[gslot, rbase + l, pl.ds(16, 16)] = b2 * v
                return 0
            lax.fori_loop(0, SUB // 16, mul_body, 0)

        def mul_and_scatter(kslot, gslot):
            for jj in range(2):
                wait_gather_sub(gslot, jj)
                mul_sub(kslot, gslot, jj)
                pltpu.async_copy(g.at[gslot, pl.ds(SUB * jj, SUB)],
                                 acc.at[ib.at[kslot, 2 + jj]],
                                 ssem[gslot], add=True)

        def wait_scatter(gslot):
            for jj in range(2):
                pltpu.make_async_copy(g.at[gslot, pl.ds(SUB * jj, SUB)],
                                      acc.at[ib.at[0, 2]],
                                      ssem[gslot]).wait()

        for p in range(N_CHUNKS):
            def zb_body(r, _):
                g[0, r, pl.ds(0, 16)] = zero16
                g[0, r, pl.ds(16, 16)] = zero16
                return 0
            lax.fori_loop(0, ZROWS, zb_body, 0)
            zcps = [
                pltpu.async_copy(
                    g.at[0, pl.ds(0, ZROWS)],
                    acc.at[pl.ds(s * ROWS_PER_TEC + t * ZROWS, ZROWS)], zsem)
                for t in range(ROWS_PER_TEC // ZROWS)
            ]
            for cp in zcps:
                cp.wait()

            for b0 in range(4):
                issue_ib(jnp.int32(b0), b0)
            prep_and_gather(jnp.int32(0), 0, 0, p)
            prep_and_gather(jnp.int32(1), 1, 1, p)
            plsc.subcore_barrier()

            def iter_body(i, _):
                for kk in range(6):
                    j = 6 * i + kk
                    gslot = kk % 3
                    if kk in (0, 1):
                        issue_ib(j + 4, (kk + 4) % 6)
                    else:
                        @pl.when(i < (N_BLK // 6 - 1))
                        def _():
                            issue_ib(j + 4, (kk + 4) % 6)
                    if kk == 0:
                        @pl.when(i > 0)
                        def _():
                            wait_scatter((kk + 2) % 3)
                    else:
                        wait_scatter((kk + 2) % 3)
                    if kk < 4:
                        prep_and_gather(j + 2, (kk + 2) % 6, (kk + 2) % 3, p)
                    else:
                        @pl.when(i < (N_BLK // 6 - 1))
                        def _():
                            prep_and_gather(j + 2, (kk + 2) % 6, (kk + 2) % 3, p)
                    mul_and_scatter(kk, gslot)
                return 0

            lax.fori_loop(0, N_BLK // 6, iter_body, 0)
            wait_scatter(2)
            plsc.subcore_barrier()

            pltpu.sync_copy(
                acc.at[pl.ds(s * ROWS_PER_TEC, ROWS_PER_TEC)],
                out_hbm.at[c, pl.ds(s * ROWS_PER_TEC, ROWS_PER_TEC),
                           pl.ds(CHUNK_W * p, CHUNK_W)])
            plsc.subcore_barrier()

    return k(u2, i2, colsb, rowsb, valsb)


def kernel(users_emb, items_emb, u_rows, u_cols, u_vals,
           i_rows, i_cols, i_vals, g_rows, g_cols, g_vals):
    del g_rows, g_cols, g_vals
    pad = NNZP - NNZ
    u2 = users_emb.reshape(N_CHUNKS * N_ROWS, CHUNK_W)
    i2 = items_emb.reshape(N_CHUNKS * N_ROWS, CHUNK_W)
    colsb = jnp.stack([jnp.pad(u_cols, (0, pad)),
                       jnp.pad(i_cols, (0, pad))]
                      ).reshape(2, N_TBLK, 2, SUB) * N_CHUNKS
    rowsb = jnp.stack([jnp.pad(u_rows, (0, pad)),
                       jnp.pad(i_rows, (0, pad))]).reshape(2, N_TBLK, 2, SUB)
    valsb = jnp.stack([jnp.pad(u_vals, (0, pad)),
                       jnp.pad(i_vals, (0, pad))]).reshape(2, N_TBLK, 2, SUB)

    bigo = _sc_spmm(u2, i2, colsb, rowsb, valsb)
    return (bigo[0, :N_ROWS], bigo[1, :N_ROWS])


# speedup vs baseline: 1.1919x; 1.0006x over previous
import functools

import jax
import jax.numpy as jnp
from jax import lax
from jax.experimental import pallas as pl
from jax.experimental.pallas import tpu as pltpu
import jax.experimental.pallas.tpu_sc as plsc

N_ROWS = 50000
LATENT = 128
CHUNK_W = 32
N_CHUNKS = LATENT // CHUNK_W
NNZ = 400000
N_SUB = 16
BLK = 256
SUB = 128
N_BLK = 102
PER_TEC = N_BLK * BLK
NNZP = PER_TEC * N_SUB
ACC_ROWS = 50176
ROWS_PER_TEC = ACC_ROWS // N_SUB
ZROWS = 112
G_RING = 3
IB_RING = 6
N_TBLK = N_SUB * N_BLK


def _sc_spmm(u2, i2, crb, valsb):
    mesh = plsc.VectorSubcoreMesh(core_axis_name="c", subcore_axis_name="s")

    @functools.partial(
        pl.kernel,
        mesh=mesh,
        compiler_params=pltpu.CompilerParams(use_tc_tiling_on_sc=False,
                                             needs_layout_passes=False),
        out_type=jax.ShapeDtypeStruct((2, ACC_ROWS, LATENT), jnp.float32),
        scratch_types=[
            pltpu.VMEM_SHARED((ACC_ROWS, CHUNK_W), jnp.float32),
            pltpu.VMEM((G_RING, BLK, CHUNK_W), jnp.float32),
            pltpu.VMEM((IB_RING, 4, SUB), jnp.int32),
            pltpu.VMEM((IB_RING, 2, SUB), jnp.float32),
        ] + [pltpu.SemaphoreType.DMA] * (G_RING + G_RING + IB_RING + 1),
    )
    def k(u2_hbm, i2_hbm, crb_hbm, valsb_hbm, out_hbm,
          acc, g, ib, vb,
          gs0, gs1, gs2, ss0, ss1, ss2,
          is0, is1, is2, is3, is4, is5, zsem):
        gsem = [gs0, gs1, gs2]
        ssem = [ss0, ss1, ss2]
        ibsem = [is0, is1, is2, is3, is4, is5]
        c = lax.axis_index("c")
        s = lax.axis_index("s")

        zero16 = jnp.zeros((16,), jnp.float32)
        tblk0 = s * N_BLK

        def issue_ib(bi, slot):
            pltpu.async_copy(crb_hbm.at[c, tblk0 + bi],
                             ib.at[slot], ibsem[slot])
            pltpu.async_copy(valsb_hbm.at[c, tblk0 + bi],
                             vb.at[slot], ibsem[slot])

        def wait_ib(slot):
            pltpu.make_async_copy(crb_hbm.at[c, tblk0],
                                  ib.at[slot], ibsem[slot]).wait()
            pltpu.make_async_copy(valsb_hbm.at[c, tblk0],
                                  vb.at[slot], ibsem[slot]).wait()

        def prep_and_gather(bi, kslot, gslot, p):
            del bi
            wait_ib(kslot)
            if p != 0:
                off = jnp.full((16,), p, jnp.int32)
                for jj in range(2):
                    for t2 in range(SUB // 16):
                        ib[kslot, jj, pl.ds(16 * t2, 16)] = (
                            ib[kslot, jj, pl.ds(16 * t2, 16)] + off)
            @pl.when(c == 0)
            def _():
                for jj in range(2):
                    pltpu.async_copy(u2_hbm.at[ib.at[kslot, jj]],
                                     g.at[gslot, pl.ds(SUB * jj, SUB)],
                                     gsem[gslot])
            @pl.when(c == 1)
            def _():
                for jj in range(2):
                    pltpu.async_copy(i2_hbm.at[ib.at[kslot, jj]],
                                     g.at[gslot, pl.ds(SUB * jj, SUB)],
                                     gsem[gslot])

        def wait_gather_sub(gslot, jj):
            pltpu.make_async_copy(u2_hbm.at[ib.at[0, 0]],
                                  g.at[gslot, pl.ds(SUB * jj, SUB)],
                                  gsem[gslot]).wait()

        def mul_sub(kslot, gslot, jj):
            def mul_body(t, _):
                vv = vb[kslot, jj, pl.ds(t * 16, 16)]
                for h in range(2):
                    rbase = jj * SUB + t * 16 + h * 8
                    loads = []
                    for l in range(8):
                        loads.append((g[gslot, rbase + l, pl.ds(0, 16)],
                                      g[gslot, rbase + l, pl.ds(16, 16)]))
                    for l in range(8):
                        v = vv[h * 8 + l]
                        a, b2 = loads[l]
                        g[gslot, rbase + l, pl.ds(0, 16)] = a * v
                        g[gslot, rbase + l, pl.ds(16, 16)] = b2 * v
                return 0
            lax.fori_loop(0, SUB // 16, mul_body, 0)

        def mul_and_scatter(kslot, gslot):
            for jj in range(2):
                wait_gather_sub(gslot, jj)
                mul_sub(kslot, gslot, jj)
                pltpu.async_copy(g.at[gslot, pl.ds(SUB * jj, SUB)],
                                 acc.at[ib.at[kslot, 2 + jj]],
                                 ssem[gslot], add=True)

        def wait_scatter(gslot):
            for jj in range(2):
                pltpu.make_async_copy(g.at[gslot, pl.ds(SUB * jj, SUB)],
                                      acc.at[ib.at[0, 2]],
                                      ssem[gslot]).wait()

        for p in range(N_CHUNKS):
            def zb_body(r, _):
                g[0, r, pl.ds(0, 16)] = zero16
                g[0, r, pl.ds(16, 16)] = zero16
                return 0
            lax.fori_loop(0, ZROWS, zb_body, 0)
            zcps = [
                pltpu.async_copy(
                    g.at[0, pl.ds(0, ZROWS)],
                    acc.at[pl.ds(s * ROWS_PER_TEC + t * ZROWS, ZROWS)], zsem)
                for t in range(ROWS_PER_TEC // ZROWS)
            ]
            for cp in zcps:
                cp.wait()

            for b0 in range(4):
                issue_ib(jnp.int32(b0), b0)
            prep_and_gather(jnp.int32(0), 0, 0, p)
            prep_and_gather(jnp.int32(1), 1, 1, p)
            plsc.subcore_barrier()

            def iter_body(i, _):
                for kk in range(6):
                    j = 6 * i + kk
                    gslot = kk % 3
                    if kk in (0, 1):
                        issue_ib(j + 4, (kk + 4) % 6)
                    else:
                        @pl.when(i < (N_BLK // 6 - 1))
                        def _():
                            issue_ib(j + 4, (kk + 4) % 6)
                    if kk == 0:
                        @pl.when(i > 0)
                        def _():
                            wait_scatter((kk + 2) % 3)
                    else:
                        wait_scatter((kk + 2) % 3)
                    if kk < 4:
                        prep_and_gather(j + 2, (kk + 2) % 6, (kk + 2) % 3, p)
                    else:
                        @pl.when(i < (N_BLK // 6 - 1))
                        def _():
                            prep_and_gather(j + 2, (kk + 2) % 6, (kk + 2) % 3, p)
                    mul_and_scatter(kk, gslot)
                return 0

            lax.fori_loop(0, N_BLK // 6, iter_body, 0)
            wait_scatter(2)
            plsc.subcore_barrier()

            pltpu.sync_copy(
                acc.at[pl.ds(s * ROWS_PER_TEC, ROWS_PER_TEC)],
                out_hbm.at[c, pl.ds(s * ROWS_PER_TEC, ROWS_PER_TEC),
                           pl.ds(CHUNK_W * p, CHUNK_W)])
            plsc.subcore_barrier()

    return k(u2, i2, crb, valsb)


def kernel(users_emb, items_emb, u_rows, u_cols, u_vals,
           i_rows, i_cols, i_vals, g_rows, g_cols, g_vals):
    del g_rows, g_cols, g_vals
    pad = NNZP - NNZ
    u2 = users_emb.reshape(N_CHUNKS * N_ROWS, CHUNK_W)
    i2 = items_emb.reshape(N_CHUNKS * N_ROWS, CHUNK_W)
    colsb = jnp.stack([jnp.pad(u_cols, (0, pad)),
                       jnp.pad(i_cols, (0, pad))]
                      ).reshape(2, N_TBLK, 2, SUB) * N_CHUNKS
    rowsb = jnp.stack([jnp.pad(u_rows, (0, pad)),
                       jnp.pad(i_rows, (0, pad))]).reshape(2, N_TBLK, 2, SUB)
    valsb = jnp.stack([jnp.pad(u_vals, (0, pad)),
                       jnp.pad(i_vals, (0, pad))]).reshape(2, N_TBLK, 2, SUB)

    crb = jnp.concatenate([colsb, rowsb], axis=2)
    bigo = _sc_spmm(u2, i2, crb, valsb)
    return (bigo[0, :N_ROWS], bigo[1, :N_ROWS])
